# baseline (speedup 1.0000x reference)
import functools

import jax
import jax.numpy as jnp
from jax import lax
from jax.experimental import pallas as pl
from jax.experimental.pallas import tpu as pltpu
from jax.experimental.pallas import tpu_sc as plsc

_MIN = -4.0
_DELTA = 0.5
_NUM_BITS = 4

_B = 64
_N = 32768
_C = 8192
_LANES = 16

_NC = 2
_NS = 16
_NW = _NC * _NS
_ROWS_PER_W = _B // _NW
_NCHUNK = _N // _C


def _tec_body(x_hbm, out_hbm, xb, ob, sem_i0, sem_i1, sem_o0, sem_o1):
    cid = lax.axis_index("c")
    sid = lax.axis_index("s")
    wid = sid * _NC + cid

    sems_i = (sem_i0, sem_i1)
    sems_o = (sem_o0, sem_o1)

    idx16 = lax.iota(jnp.int32, 16)
    tabs = [((idx16 >> (3 - k)) & 1).astype(jnp.float32) for k in range(4)]

    def compute(s):
        @plsc.parallel_loop(0, _C, step=_LANES, unroll=8)
        def body(i):
            sl = pl.ds(i, _LANES)
            v = xb[s, sl]
            t = (v - _MIN) * (1.0 / _DELTA)
            u = jnp.minimum(jnp.maximum(t, 0.0), 15.0)
            ti = u.astype(jnp.int32)
            d = jnp.where(ti.astype(jnp.float32) != u, ti + 1, ti)
            for k in range(_NUM_BITS):
                ob[s, k, sl] = jnp.take_along_axis(
                    tabs[k], d, axis=0, mode="promise_in_bounds")

    chunks = [(r, j * _C) for r in range(_ROWS_PER_W) for j in range(_NCHUNK)]
    n = len(chunks)

    def row(r):
        return wid * _ROWS_PER_W + r

    in_d = {}
    out_d = {}
    for g, (r, col) in enumerate(chunks):
        s = g & 1
        if g == 0:
            in_d[0] = pltpu.async_copy(
                x_hbm.at[row(r), pl.ds(col, _C)], xb.at[0], sems_i[0])
        if g + 1 < n:
            r2, col2 = chunks[g + 1]
            in_d[g + 1] = pltpu.async_copy(
                x_hbm.at[row(r2), pl.ds(col2, _C)], xb.at[1 - s], sems_i[1 - s])
        in_d[g].wait()
        if g >= 2:
            for h in out_d[g - 2]:
                h.wait()
        compute(s)
        out_d[g] = [
            pltpu.async_copy(
                ob.at[s, k], out_hbm.at[row(r), k, pl.ds(col, _C)], sems_o[s])
            for k in range(_NUM_BITS)
        ]
    for g in (n - 2, n - 1):
        for h in out_d[g]:
            h.wait()


@jax.jit
def _disperse(x):
    mesh = plsc.VectorSubcoreMesh(core_axis_name="c", subcore_axis_name="s")
    f = functools.partial(
        pl.kernel,
        mesh=mesh,
        out_type=jax.ShapeDtypeStruct((_B, _NUM_BITS, _N), jnp.float32),
        scratch_types=[
            pltpu.VMEM((2, _C), jnp.float32),
            pltpu.VMEM((2, _NUM_BITS, _C), jnp.float32),
            pltpu.SemaphoreType.DMA,
            pltpu.SemaphoreType.DMA,
            pltpu.SemaphoreType.DMA,
            pltpu.SemaphoreType.DMA,
        ],
    )(_tec_body)
    return f(x)


def kernel(x, unpacked):
    del unpacked
    return _disperse(x)


# speedup vs baseline: 1.2813x; 1.2813x over previous
import functools

import jax
import jax.numpy as jnp
from jax import lax
from jax.experimental import pallas as pl
from jax.experimental.pallas import tpu as pltpu
from jax.experimental.pallas import tpu_sc as plsc

_MIN = -4.0
_DELTA = 0.5
_NUM_BITS = 4

_B = 64
_N = 32768
_C = 8192
_LANES = 16

_NC = 2
_NS = 16
_NW = _NC * _NS
_ROWS_PER_W = _B // _NW
_NCHUNK = _N // _C


def _tec_body(x_hbm, out_hbm, xb, ob, sem_i0, sem_i1, sem_o0, sem_o1):
    cid = lax.axis_index("c")
    sid = lax.axis_index("s")
    wid = sid * _NC + cid

    sems_i = (sem_i0, sem_i1)
    sems_o = (sem_o0, sem_o1)

    idx16 = lax.iota(jnp.int32, 16)
    tabs = [((idx16 >> (3 - k)) & 1).astype(jnp.float32) for k in range(4)]

    def compute(s):
        @plsc.parallel_loop(0, _C, step=_LANES, unroll=4)
        def body(i):
            sl = pl.ds(i, _LANES)
            v = xb[s, sl]
            t = (v - _MIN) * (1.0 / _DELTA)
            u = jnp.minimum(jnp.maximum(t, 0.0), 15.0)
            ti = u.astype(jnp.int32)
            d = jnp.where(ti.astype(jnp.float32) != u, ti + 1, ti)
            for k in range(_NUM_BITS):
                ob[s, k, sl] = jnp.take_along_axis(
                    tabs[k], d, axis=0, mode="promise_in_bounds")

    chunks = [(r, j * _C) for r in range(_ROWS_PER_W) for j in range(_NCHUNK)]
    n = len(chunks)

    def row(r):
        return wid * _ROWS_PER_W + r

    in_d = {}
    out_d = {}
    for g, (r, col) in enumerate(chunks):
        s = g & 1
        if g == 0:
            in_d[0] = pltpu.async_copy(
                x_hbm.at[row(r), pl.ds(col, _C)], xb.at[0], sems_i[0])
        if g + 1 < n:
            r2, col2 = chunks[g + 1]
            in_d[g + 1] = pltpu.async_copy(
                x_hbm.at[row(r2), pl.ds(col2, _C)], xb.at[1 - s], sems_i[1 - s])
        in_d[g].wait()
        if g >= 2:
            for h in out_d[g - 2]:
                h.wait()
        compute(s)
        out_d[g] = [
            pltpu.async_copy(
                ob.at[s], out_hbm.at[row(r), :, pl.ds(col, _C)], sems_o[s])
        ]
    for g in (n - 2, n - 1):
        for h in out_d[g]:
            h.wait()


@jax.jit
def _disperse(x):
    mesh = plsc.VectorSubcoreMesh(core_axis_name="c", subcore_axis_name="s")
    f = functools.partial(
        pl.kernel,
        mesh=mesh,
        out_type=jax.ShapeDtypeStruct((_B, _NUM_BITS, _N), jnp.float32),
        scratch_types=[
            pltpu.VMEM((2, _C), jnp.float32),
            pltpu.VMEM((2, _NUM_BITS, _C), jnp.float32),
            pltpu.SemaphoreType.DMA,
            pltpu.SemaphoreType.DMA,
            pltpu.SemaphoreType.DMA,
            pltpu.SemaphoreType.DMA,
        ],
    )(_tec_body)
    return f(x)


def kernel(x, unpacked):
    del unpacked
    return _disperse(x)
